# baseline (speedup 1.0000x reference)
import functools

import jax
import jax.numpy as jnp
from jax import lax
from jax.experimental import pallas as pl
from jax.experimental.pallas import tpu as pltpu
from jax.experimental.pallas import tpu_sc as plsc

N_EDGE = 10000
N_NODE = 320000
DIM = 128
NCORES = 2
NSUB = 16
NTILES = NCORES * NSUB
RPT = N_NODE // NTILES
CHUNK = 40
NCH = RPT // CHUNK
NCHH = NCH // 2
HRPT = RPT // 2
NH = N_NODE // 2
NEP = 10240
EPT = NEP // NSUB

_MESH = plsc.VectorSubcoreMesh(core_axis_name="c", subcore_axis_name="s")


def _seg_body(nch, rpt, src, idxp, zeros_e, out, idx_v, b0, b1, acc, s0, s1):
    cid = lax.axis_index("c")
    sid = lax.axis_index("s")
    wid = sid * NCORES + cid
    pltpu.sync_copy(zeros_e.at[pl.ds(sid * EPT, EPT)], acc.at[pl.ds(sid * EPT, EPT)])
    pltpu.sync_copy(idxp.at[wid], idx_v)
    plsc.subcore_barrier()
    base = wid * rpt

    def chunk2(i, carry):
        c = 2 * i
        cp0 = pltpu.async_copy(src.at[pl.ds(base + c * CHUNK, CHUNK)], b0, s0)
        cp1 = pltpu.async_copy(src.at[pl.ds(base + (c + 1) * CHUNK, CHUNK)], b1, s1)
        cp0.wait()
        pltpu.sync_copy(b0, acc.at[idx_v.at[c]], add=True)
        cp1.wait()
        pltpu.sync_copy(b1, acc.at[idx_v.at[c + 1]], add=True)
        return carry

    lax.fori_loop(0, nch // 2, chunk2, 0)
    if nch % 2:
        last = nch - 1
        pltpu.async_copy(src.at[pl.ds(base + last * CHUNK, CHUNK)], b0, s0).wait()
        pltpu.sync_copy(b0, acc.at[idx_v.at[last]], add=True)
    plsc.subcore_barrier()
    pltpu.sync_copy(acc.at[pl.ds(sid * EPT, EPT)], out.at[cid, pl.ds(sid * EPT, EPT)])


def _make_seg(nch, rpt):
    return pl.kernel(
        functools.partial(_seg_body, nch, rpt),
        out_type=jax.ShapeDtypeStruct((NCORES, NEP, DIM), jnp.float32),
        mesh=_MESH,
        scratch_types=[
            pltpu.VMEM((nch, CHUNK), jnp.int32),
            pltpu.VMEM((CHUNK, DIM), jnp.float32),
            pltpu.VMEM((CHUNK, DIM), jnp.float32),
            pltpu.VMEM_SHARED((NEP, DIM), jnp.float32),
            pltpu.SemaphoreType.DMA,
            pltpu.SemaphoreType.DMA,
        ],
    )


_seg_full = _make_seg(NCH, RPT)
_seg_half = _make_seg(NCHH, HRPT)


def _cnt_body(ones_h, idxp, zeros_e, out, idx_v, ones_v, acc):
    cid = lax.axis_index("c")
    sid = lax.axis_index("s")
    wid = sid * NCORES + cid
    pltpu.sync_copy(zeros_e.at[pl.ds(sid * EPT, EPT)], acc.at[pl.ds(sid * EPT, EPT)])
    pltpu.sync_copy(idxp.at[wid], idx_v)
    pltpu.sync_copy(ones_h, ones_v)
    plsc.subcore_barrier()

    def chunk(c, carry):
        pltpu.sync_copy(ones_v, acc.at[idx_v.at[c]], add=True)
        return carry

    lax.fori_loop(0, NCH, chunk, 0)
    plsc.subcore_barrier()
    pltpu.sync_copy(acc.at[pl.ds(sid * EPT, EPT)], out.at[cid, pl.ds(sid * EPT, EPT)])


_cnt_sum = pl.kernel(
    _cnt_body,
    out_type=jax.ShapeDtypeStruct((NCORES, NEP, DIM), jnp.float32),
    mesh=_MESH,
    scratch_types=[
        pltpu.VMEM((NCH, CHUNK), jnp.int32),
        pltpu.VMEM((CHUNK, DIM), jnp.float32),
        pltpu.VMEM_SHARED((NEP, DIM), jnp.float32),
    ],
)


def _gather_body(g, idxp, out, idx_v, b0, b1, s0, s1):
    cid = lax.axis_index("c")
    sid = lax.axis_index("s")
    wid = sid * NCORES + cid
    pltpu.sync_copy(idxp.at[wid], idx_v)
    base = wid * HRPT

    def chunk2(i, carry):
        c = 2 * i
        cp0 = pltpu.async_copy(g.at[idx_v.at[c]], b0, s0)
        cp1 = pltpu.async_copy(g.at[idx_v.at[c + 1]], b1, s1)
        cp0.wait()
        pltpu.sync_copy(b0, out.at[pl.ds(base + c * CHUNK, CHUNK)])
        cp1.wait()
        pltpu.sync_copy(b1, out.at[pl.ds(base + (c + 1) * CHUNK, CHUNK)])
        return carry

    lax.fori_loop(0, NCHH // 2, chunk2, 0)
    last = NCHH - 1
    pltpu.async_copy(g.at[idx_v.at[last]], b0, s0).wait()
    pltpu.sync_copy(b0, out.at[pl.ds(base + last * CHUNK, CHUNK)])


_gather_half = pl.kernel(
    _gather_body,
    out_type=jax.ShapeDtypeStruct((NH, DIM), jnp.float32),
    mesh=_MESH,
    scratch_types=[
        pltpu.VMEM((NCHH, CHUNK), jnp.int32),
        pltpu.VMEM((CHUNK, DIM), jnp.float32),
        pltpu.VMEM((CHUNK, DIM), jnp.float32),
        pltpu.SemaphoreType.DMA,
        pltpu.SemaphoreType.DMA,
    ],
)


def _edge_block(nsp, he_ref, *refs):
    s_refs = refs[:2 * nsp]
    c0_ref, c1_ref, vwt_ref, vwb_ref, vb_ref, ewt_ref, heo_ref, g_ref = refs[2 * nsp:]
    cnt = jnp.maximum(c0_ref[...] + c1_ref[...], 1.0)
    s = s_refs[0][...]
    for r in s_refs[1:]:
        s = s + r[...]
    agg = s / cnt
    he = jnp.maximum(
        jnp.dot(he_ref[...], vwt_ref[...], preferred_element_type=jnp.float32)
        + jnp.dot(agg, vwb_ref[...], preferred_element_type=jnp.float32)
        + vb_ref[...], 0.0)
    heo_ref[...] = he
    g_ref[...] = jnp.dot(he, ewt_ref[...], preferred_element_type=jnp.float32)


def _edge_update(he, sps, cp, vwt, vwb, vb, ewt):
    blk = 2000
    grid = N_EDGE // blk
    srcs = [s[i] for s in sps for i in range(NCORES)]
    row = pl.BlockSpec((blk, DIM), lambda i: (i, 0))
    full = pl.BlockSpec((DIM, DIM), lambda i: (0, 0))
    return pl.pallas_call(
        functools.partial(_edge_block, len(sps)),
        grid=(grid,),
        in_specs=[row] * (1 + len(srcs) + 2) + [full, full,
                  pl.BlockSpec((1, DIM), lambda i: (0, 0)), full],
        out_specs=[row, row],
        out_shape=[
            jax.ShapeDtypeStruct((N_EDGE, DIM), jnp.float32),
            jax.ShapeDtypeStruct((N_EDGE, DIM), jnp.float32),
        ],
    )(he, *srcs, cp[0], cp[1], vwt, vwb, vb, ewt)


def _node_block(ow, hn_ref, gg_ref, ewb_ref, eb_ref, sel_ref, out_ref):
    gg = gg_ref[0]
    if ow != DIM:
        gg = jnp.dot(gg, sel_ref[...], preferred_element_type=jnp.float32)
    out_ref[...] = jnp.maximum(
        jnp.dot(hn_ref[0], ewb_ref[...], preferred_element_type=jnp.float32)
        + gg + eb_ref[...], 0.0)[None]


def _node_update(hn3d, p_off, gg, ewb, eb, sel, ow):
    blk = 1000
    nb = HRPT // blk
    gg3 = gg.reshape(NTILES, HRPT, DIM)
    out = pl.pallas_call(
        functools.partial(_node_block, ow),
        grid=(NTILES, nb),
        in_specs=[
            pl.BlockSpec((1, blk, DIM), lambda t, j: (t, j + p_off, 0)),
            pl.BlockSpec((1, blk, DIM), lambda t, j: (t, j, 0)),
            pl.BlockSpec((DIM, ow), lambda t, j: (0, 0)),
            pl.BlockSpec((1, ow), lambda t, j: (0, 0)),
            pl.BlockSpec((DIM, ow), lambda t, j: (0, 0)),
        ],
        out_specs=pl.BlockSpec((1, blk, ow), lambda t, j: (t, j, 0)),
        out_shape=jax.ShapeDtypeStruct((NTILES, HRPT, ow), jnp.float32),
    )(hn3d, gg3, ewb, eb, sel)
    return out.reshape(NH, ow)


def kernel(hyperedge, hyper_node, ve_affiliation,
           v2e_W0, v2e_b0, v2e_W1, v2e_b1, v2e_W2, v2e_b2,
           e2v_W0, e2v_b0, e2v_W1, e2v_b1, e2v_W2, e2v_b2):
    idx = ve_affiliation[0]
    idx3d = idx.reshape(NTILES, NCH, CHUNK)
    idxA = idx3d[:, :NCHH]
    idxB = idx3d[:, NCHH:]
    zeros_e = jnp.zeros((NEP, DIM), jnp.float32)
    ones_r = jnp.ones((CHUNK, DIM), jnp.float32)

    vW = ((v2e_W0[:DIM], v2e_W0[DIM:], v2e_b0.reshape(1, DIM)),
          (v2e_W1[:DIM], v2e_W1[DIM:], v2e_b1.reshape(1, DIM)),
          (v2e_W2[:DIM], v2e_W2[DIM:], v2e_b2.reshape(1, DIM)))
    eW = ((e2v_W0[:DIM], e2v_W0[DIM:], e2v_b0.reshape(1, DIM)),
          (e2v_W1[:DIM], e2v_W1[DIM:], e2v_b1.reshape(1, DIM)))
    e2t_pad = jnp.pad(e2v_W2[:DIM], ((0, 0), (0, DIM - 1)))
    e2b = e2v_W2[DIM:]
    sel128 = jnp.eye(DIM, dtype=jnp.float32)
    sel1 = jnp.eye(DIM, 1, dtype=jnp.float32)

    he = hyperedge
    hnA = hnB = None
    hn0_3d = hyper_node.reshape(NTILES, RPT, DIM)
    cp = _cnt_sum(ones_r, idx3d, zeros_e)[:, :N_EDGE]
    sps = [_seg_full(hyper_node, idx3d, zeros_e)[:, :N_EDGE]]
    for l in range(3):
        ewt = eW[l][0] if l < 2 else e2t_pad
        ewb = eW[l][1] if l < 2 else e2b
        ebias = eW[l][2] if l < 2 else e2v_b2.reshape(1, 1)
        sel = sel128 if l < 2 else sel1
        ow = DIM if l < 2 else 1
        he, g = _edge_update(he, sps, cp, vW[l][0], vW[l][1], vW[l][2], ewt)
        ggA = _gather_half(g, idxA)
        ggB = _gather_half(g, idxB)
        if l == 0:
            newA = _node_update(hn0_3d, 0, ggA, ewb, ebias, sel, ow)
            newB = _node_update(hn0_3d, NCHH * CHUNK // 1000, ggB, ewb, ebias,
                                sel, ow)
        else:
            newA = _node_update(hnA.reshape(NTILES, HRPT, DIM), 0, ggA, ewb,
                                ebias, sel, ow)
            newB = _node_update(hnB.reshape(NTILES, HRPT, DIM), 0, ggB, ewb,
                                ebias, sel, ow)
        if l < 2:
            spA = _seg_half(newA, idxA, zeros_e)[:, :N_EDGE]
            spB = _seg_half(newB, idxB, zeros_e)[:, :N_EDGE]
            sps = [spA, spB]
        hnA, hnB = newA, newB
    hn_out = jnp.concatenate(
        [hnA.reshape(NTILES, HRPT, 1), hnB.reshape(NTILES, HRPT, 1)],
        axis=1).reshape(N_NODE, 1)
    return (he, hn_out)


# speedup vs baseline: 1.1612x; 1.1612x over previous
import functools

import jax
import jax.numpy as jnp
from jax import lax
from jax.experimental import pallas as pl
from jax.experimental.pallas import tpu as pltpu
from jax.experimental.pallas import tpu_sc as plsc

N_EDGE = 10000
N_NODE = 320000
DIM = 128
NCORES = 2
NSUB = 16
NTILES = NCORES * NSUB
RPT = N_NODE // NTILES
CHUNK = 80
NCH = RPT // CHUNK
NEP = 10240
EPT = NEP // NSUB

_MESH = plsc.VectorSubcoreMesh(core_axis_name="c", subcore_axis_name="s")


def _seg_body(src, idx3d, zeros_e, out, idx_v, b0, b1, acc, s0, s1):
    cid = lax.axis_index("c")
    sid = lax.axis_index("s")
    wid = sid * NCORES + cid
    pltpu.sync_copy(zeros_e.at[pl.ds(sid * EPT, EPT)], acc.at[pl.ds(sid * EPT, EPT)])
    pltpu.sync_copy(idx3d.at[wid], idx_v)
    plsc.subcore_barrier()
    base = wid * RPT
    bufs = (b0, b1)
    sems = (s0, s1)

    def chunk2(i, carry):
        c = 2 * i
        cps = [pltpu.async_copy(
            src.at[pl.ds(base + (c + k) * CHUNK, CHUNK)], bufs[k], sems[k])
            for k in range(2)]
        for k in range(2):
            cps[k].wait()
            pltpu.sync_copy(bufs[k], acc.at[idx_v.at[c + k]], add=True)
        return carry

    lax.fori_loop(0, NCH // 2, chunk2, 0)
    last = NCH - 1
    pltpu.async_copy(src.at[pl.ds(base + last * CHUNK, CHUNK)], bufs[0], sems[0]).wait()
    pltpu.sync_copy(bufs[0], acc.at[idx_v.at[last]], add=True)
    plsc.subcore_barrier()
    pltpu.sync_copy(acc.at[pl.ds(sid * EPT, EPT)], out.at[cid, pl.ds(sid * EPT, EPT)])


def _make_seg():
    return pl.kernel(
        _seg_body,
        out_type=jax.ShapeDtypeStruct((NCORES, NEP, DIM), jnp.float32),
        mesh=_MESH,
        scratch_types=[
            pltpu.VMEM((NCH, CHUNK), jnp.int32),
            pltpu.VMEM((CHUNK, DIM), jnp.float32),
            pltpu.VMEM((CHUNK, DIM), jnp.float32),
            pltpu.VMEM_SHARED((NEP, DIM), jnp.float32),
            pltpu.SemaphoreType.DMA,
            pltpu.SemaphoreType.DMA,
        ],
    )


_seg_sum = _make_seg()


def _cnt_body(ones_h, idx3d, zeros_e, out, idx_v, ones_v, acc):
    cid = lax.axis_index("c")
    sid = lax.axis_index("s")
    wid = sid * NCORES + cid
    pltpu.sync_copy(zeros_e.at[pl.ds(sid * EPT, EPT)], acc.at[pl.ds(sid * EPT, EPT)])
    pltpu.sync_copy(idx3d.at[wid], idx_v)
    pltpu.sync_copy(ones_h, ones_v)
    plsc.subcore_barrier()

    def chunk(c, carry):
        pltpu.sync_copy(ones_v, acc.at[idx_v.at[c]], add=True)
        return carry

    lax.fori_loop(0, NCH, chunk, 0)
    plsc.subcore_barrier()
    pltpu.sync_copy(acc.at[pl.ds(sid * EPT, EPT)], out.at[cid, pl.ds(sid * EPT, EPT)])


_cnt_sum = pl.kernel(
    _cnt_body,
    out_type=jax.ShapeDtypeStruct((NCORES, NEP, DIM), jnp.float32),
    mesh=_MESH,
    scratch_types=[
        pltpu.VMEM((NCH, CHUNK), jnp.int32),
        pltpu.VMEM((CHUNK, DIM), jnp.float32),
        pltpu.VMEM_SHARED((NEP, DIM), jnp.float32),
    ],
)


def _gather_body(g, idx3d, out, idx_v, b0, b1, b2, b3, s0, s1, s2, s3):
    cid = lax.axis_index("c")
    sid = lax.axis_index("s")
    wid = sid * NCORES + cid
    pltpu.sync_copy(idx3d.at[wid], idx_v)
    base = wid * RPT
    bufs = (b0, b1, b2, b3)
    sems = (s0, s1, s2, s3)

    def chunk4(i, carry):
        c = 4 * i
        cps = [pltpu.async_copy(g.at[idx_v.at[c + k]], bufs[k], sems[k])
               for k in range(4)]
        for k in range(4):
            cps[k].wait()
            pltpu.sync_copy(bufs[k], out.at[pl.ds(base + (c + k) * CHUNK, CHUNK)])
        return carry

    lax.fori_loop(0, NCH // 4, chunk4, 0)
    last = NCH - 1
    pltpu.async_copy(g.at[idx_v.at[last]], bufs[0], sems[0]).wait()
    pltpu.sync_copy(bufs[0], out.at[pl.ds(base + last * CHUNK, CHUNK)])


_gather = pl.kernel(
    _gather_body,
    out_type=jax.ShapeDtypeStruct((N_NODE, DIM), jnp.float32),
    mesh=_MESH,
    scratch_types=[
        pltpu.VMEM((NCH, CHUNK), jnp.int32),
        pltpu.VMEM((CHUNK, DIM), jnp.float32),
        pltpu.VMEM((CHUNK, DIM), jnp.float32),
        pltpu.VMEM((CHUNK, DIM), jnp.float32),
        pltpu.VMEM((CHUNK, DIM), jnp.float32),
        pltpu.SemaphoreType.DMA,
        pltpu.SemaphoreType.DMA,
        pltpu.SemaphoreType.DMA,
        pltpu.SemaphoreType.DMA,
    ],
)


def _edge_block(he_ref, s0_ref, s1_ref, c0_ref, c1_ref, vwt_ref, vwb_ref,
                vb_ref, ewt_ref, heo_ref, g_ref):
    cnt = jnp.maximum(c0_ref[...] + c1_ref[...], 1.0)
    agg = (s0_ref[...] + s1_ref[...]) / cnt
    he = jnp.maximum(
        jnp.dot(he_ref[...], vwt_ref[...], preferred_element_type=jnp.float32)
        + jnp.dot(agg, vwb_ref[...], preferred_element_type=jnp.float32)
        + vb_ref[...], 0.0)
    heo_ref[...] = he
    g_ref[...] = jnp.dot(he, ewt_ref[...], preferred_element_type=jnp.float32)


def _edge_update(he, sp, cp, vwt, vwb, vb, ewt):
    blk = 2000
    grid = N_EDGE // blk
    return pl.pallas_call(
        _edge_block,
        grid=(grid,),
        in_specs=[
            pl.BlockSpec((blk, DIM), lambda i: (i, 0)),
            pl.BlockSpec((blk, DIM), lambda i: (i, 0)),
            pl.BlockSpec((blk, DIM), lambda i: (i, 0)),
            pl.BlockSpec((blk, DIM), lambda i: (i, 0)),
            pl.BlockSpec((blk, DIM), lambda i: (i, 0)),
            pl.BlockSpec((DIM, DIM), lambda i: (0, 0)),
            pl.BlockSpec((DIM, DIM), lambda i: (0, 0)),
            pl.BlockSpec((1, DIM), lambda i: (0, 0)),
            pl.BlockSpec((DIM, DIM), lambda i: (0, 0)),
        ],
        out_specs=[
            pl.BlockSpec((blk, DIM), lambda i: (i, 0)),
            pl.BlockSpec((blk, DIM), lambda i: (i, 0)),
        ],
        out_shape=[
            jax.ShapeDtypeStruct((N_EDGE, DIM), jnp.float32),
            jax.ShapeDtypeStruct((N_EDGE, DIM), jnp.float32),
        ],
    )(he, sp[0], sp[1], cp[0], cp[1], vwt, vwb, vb, ewt)


def _node_block(ow, hn_ref, gg_ref, ewb_ref, eb_ref, sel_ref, out_ref):
    gg = gg_ref[...]
    if ow != DIM:
        gg = jnp.dot(gg, sel_ref[...], preferred_element_type=jnp.float32)
    out_ref[...] = jnp.maximum(
        jnp.dot(hn_ref[...], ewb_ref[...], preferred_element_type=jnp.float32)
        + gg + eb_ref[...], 0.0)


def _node_update(hn, gg, ewb, eb, sel, ow):
    blk = 2000
    grid = N_NODE // blk
    return pl.pallas_call(
        functools.partial(_node_block, ow),
        grid=(grid,),
        in_specs=[
            pl.BlockSpec((blk, DIM), lambda i: (i, 0)),
            pl.BlockSpec((blk, DIM), lambda i: (i, 0)),
            pl.BlockSpec((DIM, ow), lambda i: (0, 0)),
            pl.BlockSpec((1, ow), lambda i: (0, 0)),
            pl.BlockSpec((DIM, ow), lambda i: (0, 0)),
        ],
        out_specs=pl.BlockSpec((blk, ow), lambda i: (i, 0)),
        out_shape=jax.ShapeDtypeStruct((N_NODE, ow), jnp.float32),
    )(hn, gg, ewb, eb, sel)


def kernel(hyperedge, hyper_node, ve_affiliation,
           v2e_W0, v2e_b0, v2e_W1, v2e_b1, v2e_W2, v2e_b2,
           e2v_W0, e2v_b0, e2v_W1, e2v_b1, e2v_W2, e2v_b2):
    idx = ve_affiliation[0]
    idx3d = idx.reshape(NTILES, NCH, CHUNK)
    zeros_e = jnp.zeros((NEP, DIM), jnp.float32)
    ones_r = jnp.ones((CHUNK, DIM), jnp.float32)

    vW = ((v2e_W0[:DIM], v2e_W0[DIM:], v2e_b0.reshape(1, DIM)),
          (v2e_W1[:DIM], v2e_W1[DIM:], v2e_b1.reshape(1, DIM)),
          (v2e_W2[:DIM], v2e_W2[DIM:], v2e_b2.reshape(1, DIM)))
    eW = ((e2v_W0[:DIM], e2v_W0[DIM:], e2v_b0.reshape(1, DIM)),
          (e2v_W1[:DIM], e2v_W1[DIM:], e2v_b1.reshape(1, DIM)))
    e2t_pad = jnp.pad(e2v_W2[:DIM], ((0, 0), (0, DIM - 1)))
    e2b = e2v_W2[DIM:]
    sel128 = jnp.eye(DIM, dtype=jnp.float32)
    sel1 = jnp.eye(DIM, 1, dtype=jnp.float32)

    he, hn = hyperedge, hyper_node
    cp = _cnt_sum(ones_r, idx3d, zeros_e)[:, :N_EDGE]
    for l in range(3):
        sp = _seg_sum(hn, idx3d, zeros_e)[:, :N_EDGE]
        if l < 2:
            he, g = _edge_update(he, sp, cp, vW[l][0], vW[l][1], vW[l][2],
                                 eW[l][0])
            gg = _gather(g, idx3d)
            hn = _node_update(hn, gg, eW[l][1], eW[l][2], sel128, DIM)
        else:
            he, g = _edge_update(he, sp, cp, vW[l][0], vW[l][1], vW[l][2],
                                 e2t_pad)
            gg = _gather(g, idx3d)
            hn = _node_update(hn, gg, e2b, e2v_b2.reshape(1, 1), sel1, 1)
    return (he, hn)
